# baseline (speedup 1.0000x reference)
import jax
import jax.numpy as jnp
from jax.experimental import pallas as pl
from jax.experimental.pallas import tpu as pltpu


def _epilogue(h, w):
    o = jax.lax.dot_general(
        h, w, (((1,), (1,)), ((), ())),
        preferred_element_type=jnp.float32)
    mean = jnp.mean(o, axis=-1, keepdims=True)
    c = o - mean
    var = jnp.mean(c * c, axis=-1, keepdims=True)
    o = c * jax.lax.rsqrt(var + 1e-5)
    return jnp.where(o >= 0, o, 0.01 * o)


def _fused_graph_conv(a_ref, x_ref, w_ref, o_ref):
    h = jnp.dot(a_ref[...], x_ref[...], preferred_element_type=jnp.float32)
    o_ref[...] = _epilogue(h, w_ref[...])


def kernel(A, X, W):
    n, k = A.shape
    d_in = X.shape[1]
    d_out = W.shape[0]
    bm = 640
    return pl.pallas_call(
        _fused_graph_conv,
        grid=(pl.cdiv(n, bm),),
        in_specs=[
            pl.BlockSpec((bm, k), lambda i: (i, 0)),
            pl.BlockSpec((k, d_in), lambda i: (0, 0)),
            pl.BlockSpec((d_out, d_in), lambda i: (0, 0)),
        ],
        out_specs=pl.BlockSpec((bm, d_out), lambda i: (i, 0)),
        out_shape=jax.ShapeDtypeStruct((n, d_out), jnp.float32),
        compiler_params=pltpu.CompilerParams(
            dimension_semantics=("parallel",),
            disable_bounds_checks=True,
            disable_semaphore_checks=True,
        ),
    )(A, X, W)


# speedup vs baseline: 1.0219x; 1.0219x over previous
import jax
import jax.numpy as jnp
from jax.experimental import pallas as pl
from jax.experimental.pallas import tpu as pltpu


def _epilogue(h, w):
    o = jax.lax.dot_general(
        h, w, (((1,), (1,)), ((), ())),
        preferred_element_type=jnp.float32)
    mean = jnp.mean(o, axis=-1, keepdims=True)
    c = o - mean
    var = jnp.mean(c * c, axis=-1, keepdims=True)
    o = c * jax.lax.rsqrt(var + 1e-5)
    return jnp.where(o >= 0, o, 0.01 * o)


def _fused_graph_conv(a0_ref, a1_ref, x_ref, w_ref, o_ref):
    bh = a0_ref.shape[0]
    h0 = jnp.dot(a0_ref[...], x_ref[...], preferred_element_type=jnp.float32)
    h1 = jnp.dot(a1_ref[...], x_ref[...], preferred_element_type=jnp.float32)
    w = w_ref[...]
    o_ref[:bh, :] = _epilogue(h0, w)
    o_ref[bh:, :] = _epilogue(h1, w)


def kernel(A, X, W):
    n, k = A.shape
    d_in = X.shape[1]
    d_out = W.shape[0]
    bm = 400 if n % 400 == 0 else n
    bh = bm // 2
    return pl.pallas_call(
        _fused_graph_conv,
        grid=(n // bm,),
        in_specs=[
            pl.BlockSpec((bh, k), lambda i: (2 * i, 0)),
            pl.BlockSpec((bh, k), lambda i: (2 * i + 1, 0)),
            pl.BlockSpec((k, d_in), lambda i: (0, 0)),
            pl.BlockSpec((d_out, d_in), lambda i: (0, 0)),
        ],
        out_specs=pl.BlockSpec((bm, d_out), lambda i: (i, 0)),
        out_shape=jax.ShapeDtypeStruct((n, d_out), jnp.float32),
        compiler_params=pltpu.CompilerParams(
            dimension_semantics=("parallel",),
            disable_bounds_checks=True,
            disable_semaphore_checks=True,
        ),
    )(A, A, X, W)
